# baseline (device time: 24685 ns/iter reference)
import jax
import jax.numpy as jnp
from jax import lax
from jax.experimental import pallas as pl
from jax.experimental.pallas import tpu as pltpu

TOKENS = 1024
DIM = 1024
VOCAB_PER_X = 8192
BLOCKS = 4
BLK = TOKENS // BLOCKS
C = 4
CH = BLK // C
H = C // 2


def kernel(ids, E):
    my_x = lax.axis_index("x")
    my_y = lax.axis_index("y")
    my_z = lax.axis_index("z")

    blk = my_y * 2 + my_z
    ids_blk = lax.dynamic_slice(ids, (blk * BLK,), (BLK,))
    loc = ids_blk - my_x * VOCAB_PER_X
    mask = (loc >= 0) & (loc < VOCAB_PER_X)
    loc_c = jnp.where(mask, loc, 0).astype(jnp.int32)
    maskcol = mask.astype(jnp.bfloat16)[:, None]
    mask_i = mask.astype(jnp.int32)

    def body(loc_ref, mask_ref, mcol_ref, e_ref, out_ref,
             part32, partb, xrecv, gsems, send_sems, recv_sems):
        x = lax.axis_index("x")
        y = lax.axis_index("y")
        z = lax.axis_index("z")
        xn = (1 - x, y, z)
        yn = (x, 1 - y, z)
        zn = (x, y, 1 - z)

        b_own = (y * 2 + z) * BLK
        b_z = (y * 2 + (1 - z)) * BLK
        b_y = ((1 - y) * 2 + z) * BLK
        b_yz = ((1 - y) * 2 + (1 - z)) * BLK

        def rdma(src, dst, sem, dev):
            return pltpu.make_async_remote_copy(
                src_ref=src, dst_ref=dst,
                send_sem=send_sems.at[sem], recv_sem=recv_sems.at[sem],
                device_id=dev, device_id_type=pl.DeviceIdType.MESH,
            )

        def oslice(base, c):
            return out_ref.at[pl.ds(base + c * CH, CH), :]

        gcps = []
        for c in range(C):
            for t in range(CH):
                i = c * CH + t
                cp = pltpu.make_async_copy(
                    e_ref.at[loc_ref[i]], part32.at[i], gsems.at[c])
                gcps.append(cp)

                @pl.when(mask_ref[i] != 0)
                def _(cp=cp):
                    cp.start()

        bar = pltpu.get_barrier_semaphore()
        for nbr in (xn, yn, zn):
            pl.semaphore_signal(
                bar, inc=1, device_id=nbr,
                device_id_type=pl.DeviceIdType.MESH,
            )
        pl.semaphore_wait(bar, 3)

        r1 = []
        for c in range(C):
            for t in range(CH):
                i = c * CH + t

                @pl.when(mask_ref[i] != 0)
                def _(cp=gcps[i]):
                    cp.wait()

            sl = pl.ds(c * CH, CH)
            partb[sl, :] = jnp.where(
                mcol_ref[sl, :] != 0,
                part32[sl, :].astype(jnp.bfloat16),
                jnp.bfloat16(0),
            )
            r1.append(rdma(partb.at[sl, :], xrecv.at[sl, :], c, xn))
            r1[c].start()

        rz = []
        ry = []
        for c in range(C):
            r1[c].wait_recv()
            out_ref[pl.ds(b_own + c * CH, CH), :] = (
                partb[pl.ds(c * CH, CH), :]
                + xrecv[pl.ds(c * CH, CH), :]
            )
            rz.append(rdma(oslice(b_own, c), oslice(b_own, c), C + c, zn))
            ry.append(rdma(oslice(b_own, c), oslice(b_own, c), 2 * C + c, yn))
            rz[c].start()
            ry[c].start()

        rzf = []
        ryf = []
        for c in range(C):
            rz[c].wait_recv()
            if c >= H:
                ryf.append(rdma(oslice(b_z, c), oslice(b_z, c),
                                3 * C + H + (c - H), yn))
                ryf[c - H].start()
            ry[c].wait_recv()
            if c < H:
                rzf.append(rdma(oslice(b_y, c), oslice(b_y, c),
                                3 * C + c, zn))
                rzf[c].start()

        for r in rzf + ryf:
            r.wait_recv()

        for r in r1 + rz + ry + rzf + ryf:
            r.wait_send()

        del b_yz

    return pl.pallas_call(
        body,
        out_shape=jax.ShapeDtypeStruct((TOKENS, DIM), jnp.bfloat16),
        in_specs=[
            pl.BlockSpec(memory_space=pltpu.SMEM),
            pl.BlockSpec(memory_space=pltpu.SMEM),
            pl.BlockSpec(memory_space=pltpu.VMEM),
            pl.BlockSpec(memory_space=pl.ANY),
        ],
        out_specs=pl.BlockSpec(memory_space=pltpu.VMEM),
        scratch_shapes=[
            pltpu.VMEM((BLK, DIM), jnp.float32),
            pltpu.VMEM((BLK, DIM), jnp.bfloat16),
            pltpu.VMEM((BLK, DIM), jnp.bfloat16),
            pltpu.SemaphoreType.DMA((C,)),
            pltpu.SemaphoreType.DMA((3 * C + 2 * H,)),
            pltpu.SemaphoreType.DMA((3 * C + 2 * H,)),
        ],
        compiler_params=pltpu.CompilerParams(collective_id=0),
    )(loc_c, mask_i, maskcol, E)


# device time: 23688 ns/iter; 1.0421x vs baseline; 1.0421x over previous
import jax
import jax.numpy as jnp
from jax import lax
from jax.experimental import pallas as pl
from jax.experimental.pallas import tpu as pltpu

TOKENS = 1024
DIM = 1024
VOCAB_PER_X = 8192
BLOCKS = 4
BLK = TOKENS // BLOCKS
C = 4
CH = BLK // C
H = C // 2


def kernel(ids, E):
    my_x = lax.axis_index("x")
    my_y = lax.axis_index("y")
    my_z = lax.axis_index("z")

    blk = my_y * 2 + my_z
    ids_blk = lax.dynamic_slice(ids, (blk * BLK,), (BLK,))
    loc = ids_blk - my_x * VOCAB_PER_X
    mask = (loc >= 0) & (loc < VOCAB_PER_X)
    loc_c = jnp.where(mask, loc, 0).astype(jnp.int32)
    maskcol = mask.astype(jnp.bfloat16)[:, None]
    def body(loc_ref, mcol_ref, e_ref, out_ref,
             part32, partb, xrecv, gsems, send_sems, recv_sems):
        x = lax.axis_index("x")
        y = lax.axis_index("y")
        z = lax.axis_index("z")
        xn = (1 - x, y, z)
        yn = (x, 1 - y, z)
        zn = (x, y, 1 - z)

        b_own = (y * 2 + z) * BLK
        b_z = (y * 2 + (1 - z)) * BLK
        b_y = ((1 - y) * 2 + z) * BLK
        b_yz = ((1 - y) * 2 + (1 - z)) * BLK

        def rdma(src, dst, sem, dev):
            return pltpu.make_async_remote_copy(
                src_ref=src, dst_ref=dst,
                send_sem=send_sems.at[sem], recv_sem=recv_sems.at[sem],
                device_id=dev, device_id_type=pl.DeviceIdType.MESH,
            )

        def oslice(base, c):
            return out_ref.at[pl.ds(base + c * CH, CH), :]

        bar = pltpu.get_barrier_semaphore()
        for nbr in (xn, yn, zn):
            pl.semaphore_signal(
                bar, inc=1, device_id=nbr,
                device_id_type=pl.DeviceIdType.MESH,
            )

        gcps = []
        for c in range(C):
            for t in range(CH):
                i = c * CH + t
                cp = pltpu.make_async_copy(
                    e_ref.at[loc_ref[i]], part32.at[i], gsems.at[c])
                gcps.append(cp)
                cp.start()

        pl.semaphore_wait(bar, 3)

        r1 = []
        for c in range(C):
            for t in range(CH):
                gcps[c * CH + t].wait()

            sl = pl.ds(c * CH, CH)
            partb[sl, :] = jnp.where(
                mcol_ref[sl, :] != 0,
                part32[sl, :].astype(jnp.bfloat16),
                jnp.bfloat16(0),
            )
            r1.append(rdma(partb.at[sl, :], xrecv.at[sl, :], c, xn))
            r1[c].start()

        rz = []
        ry = []
        for c in range(C):
            r1[c].wait_recv()
            out_ref[pl.ds(b_own + c * CH, CH), :] = (
                partb[pl.ds(c * CH, CH), :]
                + xrecv[pl.ds(c * CH, CH), :]
            )
            rz.append(rdma(oslice(b_own, c), oslice(b_own, c), C + c, zn))
            ry.append(rdma(oslice(b_own, c), oslice(b_own, c), 2 * C + c, yn))
            rz[c].start()
            ry[c].start()

        rzf = []
        ryf = []
        for c in range(C):
            rz[c].wait_recv()
            if c >= H:
                ryf.append(rdma(oslice(b_z, c), oslice(b_z, c),
                                3 * C + H + (c - H), yn))
                ryf[c - H].start()
            ry[c].wait_recv()
            if c < H:
                rzf.append(rdma(oslice(b_y, c), oslice(b_y, c),
                                3 * C + c, zn))
                rzf[c].start()

        for r in rzf + ryf:
            r.wait_recv()

        for r in r1 + rz + ry + rzf + ryf:
            r.wait_send()

        del b_yz

    return pl.pallas_call(
        body,
        out_shape=jax.ShapeDtypeStruct((TOKENS, DIM), jnp.bfloat16),
        in_specs=[
            pl.BlockSpec(memory_space=pltpu.SMEM),
            pl.BlockSpec(memory_space=pltpu.VMEM),
            pl.BlockSpec(memory_space=pl.ANY),
        ],
        out_specs=pl.BlockSpec(memory_space=pltpu.VMEM),
        scratch_shapes=[
            pltpu.VMEM((BLK, DIM), jnp.float32),
            pltpu.VMEM((BLK, DIM), jnp.bfloat16),
            pltpu.VMEM((BLK, DIM), jnp.bfloat16),
            pltpu.SemaphoreType.DMA((C,)),
            pltpu.SemaphoreType.DMA((3 * C + 2 * H,)),
            pltpu.SemaphoreType.DMA((3 * C + 2 * H,)),
        ],
        compiler_params=pltpu.CompilerParams(collective_id=0),
    )(loc_c, maskcol, E)
